# baseline (speedup 1.0000x reference)
import functools

import jax
import jax.numpy as jnp
from jax import lax
from jax.experimental import pallas as pl
from jax.experimental.pallas import tpu as pltpu
from jax.experimental.pallas import tpu_sc as plsc

_B, _L, _H = 16, 512, 128
_STEPS = 3
_NC, _NS = 2, 16
_NW = _NC * _NS
_ROWS_PER_W = _B * _L // _NW
_CHUNK = 128
_NCH = _ROWS_PER_W // _CHUNK
_RC = 128
_NRC = _L // _RC


def _sc_gather(table, idx2d):
    mesh = plsc.VectorSubcoreMesh(core_axis_name="c", subcore_axis_name="s")

    @functools.partial(
        pl.kernel,
        mesh=mesh,
        out_type=jax.ShapeDtypeStruct((_B * _L, _H), jnp.float32),
        scratch_types=[
            pltpu.VMEM((_NCH, _CHUNK), jnp.int32),
            pltpu.VMEM((_ROWS_PER_W, _H), jnp.float32),
            pltpu.SemaphoreType.DMA,
        ],
    )
    def gather_k(table_hbm, idx_hbm, out_hbm, idx_v, rows_v, sem):
        wid = lax.axis_index("s") * _NC + lax.axis_index("c")
        pltpu.sync_copy(idx_hbm.at[pl.ds(wid * _NCH, _NCH)], idx_v)
        copies = [
            pltpu.async_copy(
                table_hbm.at[idx_v.at[j]],
                rows_v.at[pl.ds(j * _CHUNK, _CHUNK)],
                sem,
            )
            for j in range(_NCH)
        ]
        for cp in copies:
            cp.wait()
        pltpu.sync_copy(rows_v, out_hbm.at[pl.ds(wid * _ROWS_PER_W, _ROWS_PER_W)])

    return gather_k(table, idx2d)


def _prep_body(a_ref, ab_ref, inv_ref):
    a = a_ref[0, :, :]
    inv_ref[0, :, :] = 1.0 / jnp.clip(
        jnp.sum(a, axis=-1, keepdims=True), 1e-6, None)
    ab_ref[0, :, :] = a.astype(jnp.bfloat16)


def _prep(adj):
    return pl.pallas_call(
        _prep_body,
        grid=(_B,),
        in_specs=[pl.BlockSpec((1, _L, _L), lambda b: (b, 0, 0))],
        out_specs=[
            pl.BlockSpec((1, _L, _L), lambda b: (b, 0, 0)),
            pl.BlockSpec((1, _L, 1), lambda b: (b, 0, 0)),
        ],
        out_shape=[
            jax.ShapeDtypeStruct((_B, _L, _L), jnp.bfloat16),
            jax.ShapeDtypeStruct((_B, _L, 1), jnp.float32),
        ],
        compiler_params=pltpu.CompilerParams(
            dimension_semantics=("arbitrary",),
        ),
    )(adj)


def _ggnn_body(len_ref, bout_ref, ab_ref, inv_ref, h_ref, wm_ref, uzr_ref,
               wzrh_ref, uh_ref, bias_ref, wout_ref, out_ref, h_scr, x_scr):
    b = pl.program_id(0)
    n = len_ref[b, 0]

    wm = wm_ref[...]
    uzr = uzr_ref[...]
    wzrh = wzrh_ref[...]
    uh = uh_ref[...]
    b_msg = bias_ref[0:1, :]
    bz = bias_ref[1:2, :]
    br = bias_ref[2:3, :]
    bh = bias_ref[3:4, :]

    def mm(x, w):
        return jnp.dot(x.astype(jnp.bfloat16), w.astype(jnp.bfloat16),
                       preferred_element_type=jnp.float32)

    h_scr[...] = jnp.zeros((_L, _H), jnp.float32)
    x_scr[...] = jnp.zeros((_L, _H), jnp.float32)
    for c in range(_NRC):
        @pl.when(c * _RC < n)
        def _():
            rows = pl.ds(c * _RC, _RC)
            maskc = ((lax.broadcasted_iota(jnp.int32, (_RC, 1), 0)
                      + c * _RC) < n).astype(jnp.float32)
            h_scr[rows, :] = h_ref[0, rows, :] * maskc

    for _ in range(_STEPS):
        for c in range(_NRC):
            @pl.when(c * _RC < n)
            def _():
                rows = pl.ds(c * _RC, _RC)
                x_scr[rows, :] = mm(h_scr[rows, :], wm)
        for c in range(_NRC):
            @pl.when(c * _RC < n)
            def _():
                rows = pl.ds(c * _RC, _RC)
                maskc = ((lax.broadcasted_iota(jnp.int32, (_RC, 1), 0)
                          + c * _RC) < n).astype(jnp.float32)
                hc = h_scr[rows, :]
                m = (mm(ab_ref[0, rows, :], x_scr[...])
                     * inv_ref[0, rows, :] + b_msg)
                g = mm(m, wzrh)
                hu = mm(hc, uzr)
                z = jax.nn.sigmoid(g[:, :_H] + hu[:, :_H] + bz)
                r = jax.nn.sigmoid(g[:, _H:2 * _H] + hu[:, _H:] + br)
                hh = jnp.tanh(g[:, 2 * _H:] + mm(r * hc, uh) + bh)
                h_scr[rows, :] = ((1.0 - z) * hc + z * hh) * maskc

    out_ref[0, :, :] = mm(h_scr[...], wout_ref[...]) + bout_ref[0]


def _tc_ggnn(ab, inv_deg, h0, seq_len, Wm, Uzr, Wzrh, Uh, biases, W_out,
             b_out):
    return pl.pallas_call(
        _ggnn_body,
        grid=(_B,),
        in_specs=[
            pl.BlockSpec(memory_space=pltpu.SMEM),
            pl.BlockSpec(memory_space=pltpu.SMEM),
            pl.BlockSpec((1, _L, _L), lambda b: (b, 0, 0)),
            pl.BlockSpec((1, _L, 1), lambda b: (b, 0, 0)),
            pl.BlockSpec((1, _L, _H), lambda b: (b, 0, 0)),
            pl.BlockSpec((_H, _H), lambda b: (0, 0)),
            pl.BlockSpec((_H, 2 * _H), lambda b: (0, 0)),
            pl.BlockSpec((_H, 3 * _H), lambda b: (0, 0)),
            pl.BlockSpec((_H, _H), lambda b: (0, 0)),
            pl.BlockSpec((4, _H), lambda b: (0, 0)),
            pl.BlockSpec((_H, 1), lambda b: (0, 0)),
        ],
        out_specs=pl.BlockSpec((1, _L, 1), lambda b: (b, 0, 0)),
        out_shape=jax.ShapeDtypeStruct((_B, _L, 1), jnp.float32),
        scratch_shapes=[
            pltpu.VMEM((_L, _H), jnp.float32),
            pltpu.VMEM((_L, _H), jnp.float32),
        ],
        compiler_params=pltpu.CompilerParams(
            dimension_semantics=("arbitrary",),
        ),
    )(seq_len, b_out, ab, inv_deg, h0, Wm, Uzr, Wzrh, Uh, biases, W_out)


def kernel(adjacent_matrix, inp_seq, inp_seq_len, embedding, W_msg, b_msg,
           Wz, Uz, bz, Wr, Ur, br, Wh, Uh, bh, W_out, b_out):
    idx2d = inp_seq.astype(jnp.int32).reshape(_NW * _NCH, _CHUNK)
    h_flat = _sc_gather(embedding, idx2d)
    h0 = h_flat.reshape(_B, _L, _H)
    ab, inv_deg = _prep(adjacent_matrix)
    biases = jnp.stack([b_msg, bz, br, bh])
    bf = jnp.bfloat16
    Uzr = jnp.concatenate([Uz, Ur], axis=1).astype(bf)
    Wzrh = jnp.concatenate([Wz, Wr, Wh], axis=1).astype(bf)
    seq_len = inp_seq_len.astype(jnp.int32).reshape(_B, 1)
    out3 = _tc_ggnn(ab, inv_deg, h0, seq_len, W_msg.astype(bf), Uzr, Wzrh,
                    Uh.astype(bf), biases, W_out.astype(bf), b_out)
    return out3.reshape(_B, _L)


# speedup vs baseline: 1.3627x; 1.3627x over previous
import functools

import jax
import jax.numpy as jnp
from jax import lax
from jax.experimental import pallas as pl
from jax.experimental.pallas import tpu as pltpu
from jax.experimental.pallas import tpu_sc as plsc

_B, _L, _H = 16, 512, 128
_STEPS = 3
_NC, _NS = 2, 16
_NW = _NC * _NS
_ROWS_PER_W = _B * _L // _NW
_CHUNK = 128
_NCH = _ROWS_PER_W // _CHUNK
_RC = 128
_NRC = _L // _RC


def _sc_gather(table, idx2d):
    mesh = plsc.VectorSubcoreMesh(core_axis_name="c", subcore_axis_name="s")

    @functools.partial(
        pl.kernel,
        mesh=mesh,
        out_type=jax.ShapeDtypeStruct((_B * _L, _H), jnp.float32),
        scratch_types=[
            pltpu.VMEM((_NCH, _CHUNK), jnp.int32),
            pltpu.VMEM((_ROWS_PER_W, _H), jnp.float32),
            pltpu.SemaphoreType.DMA,
        ],
    )
    def gather_k(table_hbm, idx_hbm, out_hbm, idx_v, rows_v, sem):
        wid = lax.axis_index("s") * _NC + lax.axis_index("c")
        pltpu.sync_copy(idx_hbm.at[pl.ds(wid * _NCH, _NCH)], idx_v)
        copies = [
            pltpu.async_copy(
                table_hbm.at[idx_v.at[j]],
                rows_v.at[pl.ds(j * _CHUNK, _CHUNK)],
                sem,
            )
            for j in range(_NCH)
        ]
        for cp in copies:
            cp.wait()
        pltpu.sync_copy(rows_v, out_hbm.at[pl.ds(wid * _ROWS_PER_W, _ROWS_PER_W)])

    return gather_k(table, idx2d)


def _prep_body(a_ref, ab_ref, inv_ref):
    a = a_ref[0, :, :]
    inv_ref[0, :, :] = 1.0 / jnp.clip(
        jnp.sum(a, axis=-1, keepdims=True), 1e-6, None)
    ab_ref[0, :, :] = a.astype(jnp.bfloat16)


def _prep(adj):
    return pl.pallas_call(
        _prep_body,
        grid=(_B,),
        in_specs=[pl.BlockSpec((1, _L, _L), lambda b: (b, 0, 0))],
        out_specs=[
            pl.BlockSpec((1, _L, _L), lambda b: (b, 0, 0)),
            pl.BlockSpec((1, _L, 1), lambda b: (b, 0, 0)),
        ],
        out_shape=[
            jax.ShapeDtypeStruct((_B, _L, _L), jnp.bfloat16),
            jax.ShapeDtypeStruct((_B, _L, 1), jnp.float32),
        ],
        compiler_params=pltpu.CompilerParams(
            dimension_semantics=("arbitrary",),
        ),
    )(adj)


_BPP = 2


def _ggnn_body(len_ref, bout_ref, ab_ref, inv_ref, h_ref, wm_ref, uzr_ref,
               wzrh_ref, uh_ref, bias_ref, wout_ref, out_ref):
    g0 = pl.program_id(0)

    wm = wm_ref[...]
    uzr = uzr_ref[...]
    wzrh = wzrh_ref[...]
    uh = uh_ref[...]
    b_msg = bias_ref[0:1, :]
    bz = bias_ref[1:2, :]
    br = bias_ref[2:3, :]
    bh = bias_ref[3:4, :]
    wout = wout_ref[...]

    def mm(x, w):
        return jnp.dot(x.astype(jnp.bfloat16), w.astype(jnp.bfloat16),
                       preferred_element_type=jnp.float32)

    for j in range(_BPP):
        n = len_ref[g0 * _BPP + j, 0]
        mask = (lax.broadcasted_iota(jnp.int32, (_L, 1), 0)
                < n).astype(jnp.float32)
        h = h_ref[j, :, :] * mask
        ab = ab_ref[j, :, :]
        inv_deg = inv_ref[j, :, :]
        for _ in range(_STEPS):
            x = mm(h, wm)
            m = mm(ab, x) * inv_deg + b_msg
            g = mm(m, wzrh)
            hu = mm(h, uzr)
            z = jax.nn.sigmoid(g[:, :_H] + hu[:, :_H] + bz)
            r = jax.nn.sigmoid(g[:, _H:2 * _H] + hu[:, _H:] + br)
            hh = jnp.tanh(g[:, 2 * _H:] + mm(r * h, uh) + bh)
            h = ((1.0 - z) * h + z * hh) * mask
        out_ref[j, :, :] = mm(h, wout) + bout_ref[0]


def _tc_ggnn(ab, inv_deg, h0, seq_len, Wm, Uzr, Wzrh, Uh, biases, W_out,
             b_out):
    return pl.pallas_call(
        _ggnn_body,
        grid=(_B // _BPP,),
        in_specs=[
            pl.BlockSpec(memory_space=pltpu.SMEM),
            pl.BlockSpec(memory_space=pltpu.SMEM),
            pl.BlockSpec((_BPP, _L, _L), lambda b: (b, 0, 0)),
            pl.BlockSpec((_BPP, _L, 1), lambda b: (b, 0, 0)),
            pl.BlockSpec((_BPP, _L, _H), lambda b: (b, 0, 0)),
            pl.BlockSpec((_H, _H), lambda b: (0, 0)),
            pl.BlockSpec((_H, 2 * _H), lambda b: (0, 0)),
            pl.BlockSpec((_H, 3 * _H), lambda b: (0, 0)),
            pl.BlockSpec((_H, _H), lambda b: (0, 0)),
            pl.BlockSpec((4, _H), lambda b: (0, 0)),
            pl.BlockSpec((_H, 1), lambda b: (0, 0)),
        ],
        out_specs=pl.BlockSpec((_BPP, _L, 1), lambda b: (b, 0, 0)),
        out_shape=jax.ShapeDtypeStruct((_B, _L, 1), jnp.float32),
        compiler_params=pltpu.CompilerParams(
            dimension_semantics=("arbitrary",),
        ),
    )(seq_len, b_out, ab, inv_deg, h0, Wm, Uzr, Wzrh, Uh, biases, W_out)


def kernel(adjacent_matrix, inp_seq, inp_seq_len, embedding, W_msg, b_msg,
           Wz, Uz, bz, Wr, Ur, br, Wh, Uh, bh, W_out, b_out):
    idx2d = inp_seq.astype(jnp.int32).reshape(_NW * _NCH, _CHUNK)
    h_flat = _sc_gather(embedding, idx2d)
    h0 = h_flat.reshape(_B, _L, _H)
    ab, inv_deg = _prep(adjacent_matrix)
    biases = jnp.stack([b_msg, bz, br, bh])
    bf = jnp.bfloat16
    Uzr = jnp.concatenate([Uz, Ur], axis=1).astype(bf)
    Wzrh = jnp.concatenate([Wz, Wr, Wh], axis=1).astype(bf)
    seq_len = inp_seq_len.astype(jnp.int32).reshape(_B, 1)
    out3 = _tc_ggnn(ab, inv_deg, h0, seq_len, W_msg.astype(bf), Uzr, Wzrh,
                    Uh.astype(bf), biases, W_out.astype(bf), b_out)
    return out3.reshape(_B, _L)


# speedup vs baseline: 1.6959x; 1.2445x over previous
import functools

import jax
import jax.numpy as jnp
from jax import lax
from jax.experimental import pallas as pl
from jax.experimental.pallas import tpu as pltpu
from jax.experimental.pallas import tpu_sc as plsc

_B, _L, _H = 16, 512, 128
_STEPS = 3
_NC, _NS = 2, 16
_NW = _NC * _NS
_ROWS_PER_W = _B * _L // _NW
_CHUNK = 128
_NCH = _ROWS_PER_W // _CHUNK


def _sc_gather(table, idx2d):
    mesh = plsc.VectorSubcoreMesh(core_axis_name="c", subcore_axis_name="s")

    @functools.partial(
        pl.kernel,
        mesh=mesh,
        out_type=jax.ShapeDtypeStruct((_B * _L, _H), jnp.float32),
        scratch_types=[
            pltpu.VMEM((_NCH, _CHUNK), jnp.int32),
            pltpu.VMEM((_ROWS_PER_W, _H), jnp.float32),
            pltpu.SemaphoreType.DMA,
        ],
    )
    def gather_k(table_hbm, idx_hbm, out_hbm, idx_v, rows_v, sem):
        wid = lax.axis_index("s") * _NC + lax.axis_index("c")
        pltpu.sync_copy(idx_hbm.at[pl.ds(wid * _NCH, _NCH)], idx_v)
        copies = [
            pltpu.async_copy(
                table_hbm.at[idx_v.at[j]],
                rows_v.at[pl.ds(j * _CHUNK, _CHUNK)],
                sem,
            )
            for j in range(_NCH)
        ]
        for cp in copies:
            cp.wait()
        pltpu.sync_copy(rows_v, out_hbm.at[pl.ds(wid * _ROWS_PER_W, _ROWS_PER_W)])

    return gather_k(table, idx2d)


def _ggnn_body(len_ref, bout_ref, a_ref, h_ref, wmzr_ref, wzrh_ref,
               uh_ref, bias_ref, wout_ref, out_ref):
    b = pl.program_id(0)
    n = len_ref[b, 0]
    mask = (lax.broadcasted_iota(jnp.int32, (_L, 1), 0) < n).astype(jnp.float32)
    h = h_ref[0, :, :] * mask
    a = a_ref[0, :, :]
    inv_deg = 1.0 / jnp.clip(jnp.sum(a, axis=-1, keepdims=True), 1e-6, None)
    ab = a.astype(jnp.bfloat16)

    wmzr = wmzr_ref[...].astype(jnp.bfloat16)
    wzrh = wzrh_ref[...].astype(jnp.bfloat16)
    uh = uh_ref[...].astype(jnp.bfloat16)
    b_msg = bias_ref[0:1, :]
    bz = bias_ref[1:2, :]
    br = bias_ref[2:3, :]
    bh = bias_ref[3:4, :]

    def mm(x, w):
        return jnp.dot(x.astype(jnp.bfloat16), w,
                       preferred_element_type=jnp.float32)

    for _ in range(_STEPS):
        c = mm(h, wmzr)
        x = c[:, :_H]
        m = mm(ab, x) * inv_deg + b_msg
        g = mm(m, wzrh)
        z = 0.5 * jnp.tanh(0.5 * (g[:, :_H] + c[:, _H:2 * _H] + bz)) + 0.5
        r = 0.5 * jnp.tanh(0.5 * (g[:, _H:2 * _H] + c[:, 2 * _H:] + br)) + 0.5
        hh = jnp.tanh(g[:, 2 * _H:] + mm(r * h, uh) + bh)
        h = ((1.0 - z) * h + z * hh) * mask

    out_ref[0, :, :] = mm(h, wout_ref[...].astype(jnp.bfloat16)) + bout_ref[0]


def _tc_ggnn(adj, h0, seq_len, Wmzr, Wzrh, Uh, biases, W_out, b_out):
    return pl.pallas_call(
        _ggnn_body,
        grid=(_B,),
        in_specs=[
            pl.BlockSpec(memory_space=pltpu.SMEM),
            pl.BlockSpec(memory_space=pltpu.SMEM),
            pl.BlockSpec((1, _L, _L), lambda b: (b, 0, 0)),
            pl.BlockSpec((1, _L, _H), lambda b: (b, 0, 0)),
            pl.BlockSpec((_H, 3 * _H), lambda b: (0, 0)),
            pl.BlockSpec((_H, 3 * _H), lambda b: (0, 0)),
            pl.BlockSpec((_H, _H), lambda b: (0, 0)),
            pl.BlockSpec((4, _H), lambda b: (0, 0)),
            pl.BlockSpec((_H, 1), lambda b: (0, 0)),
        ],
        out_specs=pl.BlockSpec((1, _L, 1), lambda b: (b, 0, 0)),
        out_shape=jax.ShapeDtypeStruct((_B, _L, 1), jnp.float32),
        compiler_params=pltpu.CompilerParams(
            dimension_semantics=("arbitrary",),
        ),
    )(seq_len, b_out, adj, h0, Wmzr, Wzrh, Uh, biases, W_out)


def kernel(adjacent_matrix, inp_seq, inp_seq_len, embedding, W_msg, b_msg,
           Wz, Uz, bz, Wr, Ur, br, Wh, Uh, bh, W_out, b_out):
    idx2d = inp_seq.astype(jnp.int32).reshape(_NW * _NCH, _CHUNK)
    h_flat = _sc_gather(embedding, idx2d)
    h0 = h_flat.reshape(_B, _L, _H)
    biases = jnp.stack([b_msg, bz, br, bh])
    Wmzr = jnp.concatenate([W_msg, Uz, Ur], axis=1)
    Wzrh = jnp.concatenate([Wz, Wr, Wh], axis=1)
    seq_len = inp_seq_len.astype(jnp.int32).reshape(_B, 1)
    out3 = _tc_ggnn(adjacent_matrix, h0, seq_len, Wmzr, Wzrh, Uh,
                    biases, W_out, b_out)
    return out3.reshape(_B, _L)


# speedup vs baseline: 1.7366x; 1.0240x over previous
import functools

import jax
import jax.numpy as jnp
from jax import lax
from jax.experimental import pallas as pl
from jax.experimental.pallas import tpu as pltpu
from jax.experimental.pallas import tpu_sc as plsc

_B, _L, _H = 16, 512, 128
_STEPS = 3
_NC, _NS = 2, 16
_NW = _NC * _NS
_ROWS_PER_W = _B * _L // _NW
_CHUNK = 128
_NCH = _ROWS_PER_W // _CHUNK


def _sc_gather(table, idx2d):
    mesh = plsc.VectorSubcoreMesh(core_axis_name="c", subcore_axis_name="s")

    @functools.partial(
        pl.kernel,
        mesh=mesh,
        out_type=jax.ShapeDtypeStruct((_B * _L, _H), jnp.float32),
        scratch_types=[
            pltpu.VMEM((_NCH, _CHUNK), jnp.int32),
            pltpu.VMEM((_ROWS_PER_W, _H), jnp.float32),
            pltpu.SemaphoreType.DMA,
        ],
    )
    def gather_k(table_hbm, idx_hbm, out_hbm, idx_v, rows_v, sem):
        wid = lax.axis_index("s") * _NC + lax.axis_index("c")
        pltpu.sync_copy(idx_hbm.at[pl.ds(wid * _NCH, _NCH)], idx_v)
        copies = [
            pltpu.async_copy(
                table_hbm.at[idx_v.at[j]],
                rows_v.at[pl.ds(j * _CHUNK, _CHUNK)],
                sem,
            )
            for j in range(_NCH)
        ]
        for cp in copies:
            cp.wait()
        pltpu.sync_copy(rows_v, out_hbm.at[pl.ds(wid * _ROWS_PER_W, _ROWS_PER_W)])

    return gather_k(table, idx2d)


_BPP = 2


def _ggnn_body(len_ref, bout_ref, a_ref, h_ref, wmzr_ref, wzrh_ref,
               uh_ref, bias_ref, wout_ref, out_ref):
    g0 = pl.program_id(0)

    wmzr = wmzr_ref[...].astype(jnp.bfloat16)
    wzrh = wzrh_ref[...].astype(jnp.bfloat16)
    uh = uh_ref[...].astype(jnp.bfloat16)
    wout = wout_ref[...].astype(jnp.bfloat16)
    b_msg = bias_ref[0:1, :]
    bz = bias_ref[1:2, :]
    br = bias_ref[2:3, :]
    bh = bias_ref[3:4, :]

    def mm(x, w):
        return jnp.dot(x.astype(jnp.bfloat16), w,
                       preferred_element_type=jnp.float32)

    for j in range(_BPP):
        n = len_ref[g0 * _BPP + j, 0]
        mask = (lax.broadcasted_iota(jnp.int32, (_L, 1), 0)
                < n).astype(jnp.float32)
        h = h_ref[j, :, :] * mask
        a = a_ref[j, :, :]
        inv_deg = 1.0 / jnp.clip(jnp.sum(a, axis=-1, keepdims=True),
                                 1e-6, None)
        ab = a.astype(jnp.bfloat16)
        for _ in range(_STEPS):
            c = mm(h, wmzr)
            x = c[:, :_H]
            m = mm(ab, x) * inv_deg + b_msg
            g = mm(m, wzrh)
            z = 0.5 * jnp.tanh(0.5 * (g[:, :_H] + c[:, _H:2 * _H] + bz)) + 0.5
            r = 0.5 * jnp.tanh(0.5 * (g[:, _H:2 * _H] + c[:, 2 * _H:] + br)) + 0.5
            hh = jnp.tanh(g[:, 2 * _H:] + mm(r * h, uh) + bh)
            h = ((1.0 - z) * h + z * hh) * mask
        out_ref[j, :, :] = mm(h, wout) + bout_ref[0]


def _tc_ggnn(adj, h0, seq_len, Wmzr, Wzrh, Uh, biases, W_out, b_out):
    return pl.pallas_call(
        _ggnn_body,
        grid=(_B // _BPP,),
        in_specs=[
            pl.BlockSpec(memory_space=pltpu.SMEM),
            pl.BlockSpec(memory_space=pltpu.SMEM),
            pl.BlockSpec((_BPP, _L, _L), lambda b: (b, 0, 0)),
            pl.BlockSpec((_BPP, _L, _H), lambda b: (b, 0, 0)),
            pl.BlockSpec((_H, 3 * _H), lambda b: (0, 0)),
            pl.BlockSpec((_H, 3 * _H), lambda b: (0, 0)),
            pl.BlockSpec((_H, _H), lambda b: (0, 0)),
            pl.BlockSpec((4, _H), lambda b: (0, 0)),
            pl.BlockSpec((_H, 1), lambda b: (0, 0)),
        ],
        out_specs=pl.BlockSpec((_BPP, _L, 1), lambda b: (b, 0, 0)),
        out_shape=jax.ShapeDtypeStruct((_B, _L, 1), jnp.float32),
        compiler_params=pltpu.CompilerParams(
            dimension_semantics=("arbitrary",),
        ),
    )(seq_len, b_out, adj, h0, Wmzr, Wzrh, Uh, biases, W_out)


def kernel(adjacent_matrix, inp_seq, inp_seq_len, embedding, W_msg, b_msg,
           Wz, Uz, bz, Wr, Ur, br, Wh, Uh, bh, W_out, b_out):
    idx2d = inp_seq.astype(jnp.int32).reshape(_NW * _NCH, _CHUNK)
    h_flat = _sc_gather(embedding, idx2d)
    h0 = h_flat.reshape(_B, _L, _H)
    biases = jnp.stack([b_msg, bz, br, bh])
    Wmzr = jnp.concatenate([W_msg, Uz, Ur], axis=1)
    Wzrh = jnp.concatenate([Wz, Wr, Wh], axis=1)
    seq_len = inp_seq_len.astype(jnp.int32).reshape(_B, 1)
    out3 = _tc_ggnn(adjacent_matrix, h0, seq_len, Wmzr, Wzrh, Uh,
                    biases, W_out, b_out)
    return out3.reshape(_B, _L)


# speedup vs baseline: 1.7539x; 1.0100x over previous
import functools

import jax
import jax.numpy as jnp
from jax import lax
from jax.experimental import pallas as pl
from jax.experimental.pallas import tpu as pltpu
from jax.experimental.pallas import tpu_sc as plsc

_B, _L, _H = 16, 512, 128
_STEPS = 3
_NC, _NS = 2, 16
_NW = _NC * _NS
_ROWS_PER_W = _B * _L // _NW
_CHUNK = 128
_NCH = _ROWS_PER_W // _CHUNK


def _sc_gather(table, idx2d):
    mesh = plsc.VectorSubcoreMesh(core_axis_name="c", subcore_axis_name="s")

    @functools.partial(
        pl.kernel,
        mesh=mesh,
        out_type=jax.ShapeDtypeStruct((_B * _L, _H), jnp.float32),
        scratch_types=[
            pltpu.VMEM((_NCH, _CHUNK), jnp.int32),
            pltpu.VMEM((_ROWS_PER_W, _H), jnp.float32),
            pltpu.SemaphoreType.DMA,
        ],
    )
    def gather_k(table_hbm, idx_hbm, out_hbm, idx_v, rows_v, sem):
        wid = lax.axis_index("s") * _NC + lax.axis_index("c")
        pltpu.sync_copy(idx_hbm.at[pl.ds(wid * _NCH, _NCH)], idx_v)
        copies = [
            pltpu.async_copy(
                table_hbm.at[idx_v.at[j]],
                rows_v.at[pl.ds(j * _CHUNK, _CHUNK)],
                sem,
            )
            for j in range(_NCH)
        ]
        for cp in copies:
            cp.wait()
        pltpu.sync_copy(rows_v, out_hbm.at[pl.ds(wid * _ROWS_PER_W, _ROWS_PER_W)])

    return gather_k(table, idx2d)


_BPP = 4


def _ggnn_body(len_ref, bout_ref, a_ref, h_ref, wmzr_ref, wzrh_ref,
               uh_ref, bias_ref, wout_ref, out_ref):
    g0 = pl.program_id(0)

    wmzr = wmzr_ref[...].astype(jnp.bfloat16)
    wzrh = wzrh_ref[...].astype(jnp.bfloat16)
    uh = uh_ref[...].astype(jnp.bfloat16)
    wout = wout_ref[...].astype(jnp.bfloat16)
    b_msg = bias_ref[0:1, :]
    bz = bias_ref[1:2, :]
    br = bias_ref[2:3, :]
    bh = bias_ref[3:4, :]

    def mm(x, w):
        return jnp.dot(x.astype(jnp.bfloat16), w,
                       preferred_element_type=jnp.float32)

    for j in range(_BPP):
        n = len_ref[g0 * _BPP + j, 0]
        mask = (lax.broadcasted_iota(jnp.int32, (_L, 1), 0)
                < n).astype(jnp.float32)
        h = h_ref[j, :, :] * mask
        a = a_ref[j, :, :]
        inv_deg = 1.0 / jnp.clip(jnp.sum(a, axis=-1, keepdims=True),
                                 1e-6, None)
        ab = a.astype(jnp.bfloat16)
        for _ in range(_STEPS):
            c = mm(h, wmzr)
            x = c[:, :_H]
            m = mm(ab, x) * inv_deg + b_msg
            g = mm(m, wzrh)
            z = 0.5 * jnp.tanh(0.5 * (g[:, :_H] + c[:, _H:2 * _H] + bz)) + 0.5
            r = 0.5 * jnp.tanh(0.5 * (g[:, _H:2 * _H] + c[:, 2 * _H:] + br)) + 0.5
            hh = jnp.tanh(g[:, 2 * _H:] + mm(r * h, uh) + bh)
            h = ((1.0 - z) * h + z * hh) * mask
        out_ref[j, :, :] = mm(h, wout) + bout_ref[0]


def _tc_ggnn(adj, h0, seq_len, Wmzr, Wzrh, Uh, biases, W_out, b_out):
    return pl.pallas_call(
        _ggnn_body,
        grid=(_B // _BPP,),
        in_specs=[
            pl.BlockSpec(memory_space=pltpu.SMEM),
            pl.BlockSpec(memory_space=pltpu.SMEM),
            pl.BlockSpec((_BPP, _L, _L), lambda b: (b, 0, 0)),
            pl.BlockSpec((_BPP, _L, _H), lambda b: (b, 0, 0)),
            pl.BlockSpec((_H, 3 * _H), lambda b: (0, 0)),
            pl.BlockSpec((_H, 3 * _H), lambda b: (0, 0)),
            pl.BlockSpec((_H, _H), lambda b: (0, 0)),
            pl.BlockSpec((4, _H), lambda b: (0, 0)),
            pl.BlockSpec((_H, 1), lambda b: (0, 0)),
        ],
        out_specs=pl.BlockSpec((_BPP, _L, 1), lambda b: (b, 0, 0)),
        out_shape=jax.ShapeDtypeStruct((_B, _L, 1), jnp.float32),
        compiler_params=pltpu.CompilerParams(
            dimension_semantics=("arbitrary",),
        ),
    )(seq_len, b_out, adj, h0, Wmzr, Wzrh, Uh, biases, W_out)


def kernel(adjacent_matrix, inp_seq, inp_seq_len, embedding, W_msg, b_msg,
           Wz, Uz, bz, Wr, Ur, br, Wh, Uh, bh, W_out, b_out):
    idx2d = inp_seq.astype(jnp.int32).reshape(_NW * _NCH, _CHUNK)
    h_flat = _sc_gather(embedding, idx2d)
    h0 = h_flat.reshape(_B, _L, _H)
    biases = jnp.stack([b_msg, bz, br, bh])
    Wmzr = jnp.concatenate([W_msg, Uz, Ur], axis=1)
    Wzrh = jnp.concatenate([Wz, Wr, Wh], axis=1)
    seq_len = inp_seq_len.astype(jnp.int32).reshape(_B, 1)
    out3 = _tc_ggnn(adjacent_matrix, h0, seq_len, Wmzr, Wzrh, Uh,
                    biases, W_out, b_out)
    return out3.reshape(_B, _L)
